# baseline (speedup 1.0000x reference)
import functools

import jax
import jax.numpy as jnp
from jax import lax
from jax.experimental import pallas as pl
from jax.experimental.pallas import tpu as pltpu
from jax.experimental.pallas import tpu_sc as plsc

_ENTITY_NUM = 100000
_DIM = 64
_MARGIN = 1.0
_BATCH = 16384

_NC = 2
_NS = 16
_NW = _NC * _NS
_NPASS = _DIM // _NW
_CI = 2048
_NCH = _BATCH // _CI
_UNROLL = 16


def _sc_partials(cur_t, cor_t, ent_t, rel_t):
    mesh = plsc.VectorSubcoreMesh(core_axis_name="c", subcore_axis_name="s")

    @functools.partial(
        pl.kernel,
        mesh=mesh,
        out_type=(
            jax.ShapeDtypeStruct((2, _DIM, 128), jnp.float32),
            jax.ShapeDtypeStruct((_NW, 2, _BATCH), jnp.float32),
        ),
        scratch_types=[
            pltpu.VMEM((1, _ENTITY_NUM), jnp.float32),
            pltpu.VMEM((2, 2, _CI), jnp.float32),
            pltpu.VMEM((2, 4, _CI), jnp.int32),
            pltpu.VMEM((2, 1, 128), jnp.float32),
            pltpu.SemaphoreType.DMA,
            pltpu.SemaphoreType.DMA,
            pltpu.SemaphoreType.DMA,
        ],
        compiler_params=pltpu.CompilerParams(
            use_tc_tiling_on_sc=True, needs_layout_passes=False),
    )
    def body(cur_hbm, cor_hbm, ent_hbm, rel_hbm, out_hbm, sp_hbm,
             row_v, vbuf, idxb, accv, sem, sps0, sps1):
        wid = lax.axis_index("s") * _NC + lax.axis_index("c")
        z16 = jnp.zeros((16,), jnp.int32)

        def idx_pairs(cols, c):
            ds = pl.ds(c * _CI, _CI)
            buf = lax.rem(c, 2)
            return [(src.at[pl.ds(col, 1), ds], idxb.at[buf, pl.ds(j, 1)])
                    for j, (src, col) in enumerate(cols)]

        def start_idx(cols, c):
            for s, t in idx_pairs(cols, c):
                pltpu.async_copy(s, t, sem)

        def wait_idx(cols, c):
            for s, t in idx_pairs(cols, c):
                pltpu.make_async_copy(s, t, sem).wait()

        def spill_pair(c):
            buf = lax.rem(c, 2)
            return (vbuf.at[pl.ds(buf, 1)],
                    sp_hbm.at[pl.ds(wid, 1), pl.ds(0, 2), pl.ds(c * _CI, _CI)])

        def on_spill_sem(c, fn):
            v, h = spill_pair(c)

            @pl.when(lax.rem(c, 2) == 0)
            def _():
                fn(v, h, sps0)

            @pl.when(lax.rem(c, 2) == 1)
            def _():
                fn(v, h, sps1)

        ecols = [(cur_hbm, 0), (cur_hbm, 2), (cor_hbm, 0), (cor_hbm, 2)]
        rcols = [(cur_hbm, 1), (cor_hbm, 1)]

        def pass_body(p, _):
            d = wid + _NW * p

            pltpu.sync_copy(ent_hbm.at[pl.ds(d, 1)], row_v)
            start_idx(ecols, 0)

            def e_chunk(c, _):
                wait_idx(ecols, c)

                @pl.when(c + 1 < _NCH)
                def _():
                    start_idx(ecols, c + 1)

                @pl.when(c >= 2)
                def _():
                    on_spill_sem(c - 2,
                                 lambda v, h, s:
                                 pltpu.make_async_copy(v, h, s).wait())

                buf = lax.rem(c, 2)

                def eb(i, _):
                    for u in range(_UNROLL):
                        off = i * (16 * _UNROLL) + u * 16
                        for l in range(2):
                            hi = idxb[buf, 2 * l, pl.ds(off, 16)]
                            ti = idxb[buf, 2 * l + 1, pl.ds(off, 16)]
                            gh = plsc.load_gather(row_v, [z16, hi])
                            gt = plsc.load_gather(row_v, [z16, ti])
                            vbuf[buf, l, pl.ds(off, 16)] = gh - gt
                    return 0

                lax.fori_loop(0, _CI // (16 * _UNROLL), eb, 0)
                on_spill_sem(c, lambda v, h, s: pltpu.async_copy(v, h, s))
                return 0

            lax.fori_loop(0, _NCH, e_chunk, 0)
            for cc in (_NCH - 2, _NCH - 1):
                on_spill_sem(cc,
                             lambda v, h, s:
                             pltpu.make_async_copy(v, h, s).wait())

            pltpu.sync_copy(rel_hbm.at[pl.ds(d, 1)], row_v)
            start_idx(rcols, 0)
            for cc in (0, 1):
                on_spill_sem(cc, lambda v, h, s: pltpu.async_copy(h, v, s))

            def r_chunk(c, accs):
                wait_idx(rcols, c)

                @pl.when(c + 1 < _NCH)
                def _():
                    start_idx(rcols, c + 1)

                on_spill_sem(c,
                             lambda v, h, s:
                             pltpu.make_async_copy(h, v, s).wait())
                buf = lax.rem(c, 2)

                def rb(i, carry):
                    a0, a1 = carry
                    for u in range(_UNROLL):
                        off = i * (16 * _UNROLL) + u * 16
                        ri = idxb[buf, 0, pl.ds(off, 16)]
                        rci = idxb[buf, 1, pl.ds(off, 16)]
                        v0 = vbuf[buf, 0, pl.ds(off, 16)] \
                            + plsc.load_gather(row_v, [z16, ri])
                        v1 = vbuf[buf, 1, pl.ds(off, 16)] \
                            + plsc.load_gather(row_v, [z16, rci])
                        a0 = a0 + v0 * v0
                        a1 = a1 + v1 * v1
                    return (a0, a1)

                accs = lax.fori_loop(0, _CI // (16 * _UNROLL), rb, accs)

                @pl.when(c + 2 < _NCH)
                def _():
                    on_spill_sem(c + 2,
                                 lambda v, h, s: pltpu.async_copy(h, v, s))

                return accs

            accs = lax.fori_loop(
                0, _NCH, r_chunk,
                (jnp.zeros((16,), jnp.float32), jnp.zeros((16,), jnp.float32)))
            for l in range(2):
                accv[l, 0, pl.ds(0, 16)] = accs[l]
                pltpu.sync_copy(accv.at[pl.ds(l, 1)],
                                out_hbm.at[pl.ds(l, 1), pl.ds(d, 1)])
            return 0

        lax.fori_loop(0, _NPASS, pass_body, 0)

    return body(cur_t, cor_t, ent_t, rel_t)[0]


def _finish(partials):
    def body(p_ref, o_ref):
        p = p_ref[:, :, 0:16]
        s = jnp.sum(p, axis=2)
        dist = jnp.sqrt(s)
        m = jnp.maximum(dist[0:1] - dist[1:2] + _MARGIN, 0.0)
        o_ref[...] = jnp.sum(m, axis=1, keepdims=True) * (1.0 / _DIM)

    return pl.pallas_call(
        body,
        out_shape=jax.ShapeDtypeStruct((1, 1), jnp.float32),
    )(partials)


@jax.jit
def kernel(current_list, corrupt_list, ent_emb, rel_emb):
    partials = _sc_partials(
        current_list.T, corrupt_list.T, ent_emb.T, rel_emb.T)
    loss = _finish(partials)
    return loss[0, 0]


# speedup vs baseline: 1.0148x; 1.0148x over previous
import functools

import jax
import jax.numpy as jnp
from jax import lax
from jax.experimental import pallas as pl
from jax.experimental.pallas import tpu as pltpu
from jax.experimental.pallas import tpu_sc as plsc

_ENTITY_NUM = 100000
_DIM = 64
_MARGIN = 1.0
_BATCH = 16384

_NC = 2
_NS = 16
_NW = _NC * _NS
_NPASS = _DIM // _NW
_CI = 2048
_NCH = _BATCH // _CI
_UNROLL = 8


def _sc_partials(cur_t, cor_t, ent_t, rel_t):
    mesh = plsc.VectorSubcoreMesh(core_axis_name="c", subcore_axis_name="s")

    @functools.partial(
        pl.kernel,
        mesh=mesh,
        out_type=(
            jax.ShapeDtypeStruct((2, _DIM, 128), jnp.float32),
            jax.ShapeDtypeStruct((_NW, 2, _BATCH), jnp.float32),
        ),
        scratch_types=[
            pltpu.VMEM((1, _ENTITY_NUM), jnp.float32),
            pltpu.VMEM((2, 2, _CI), jnp.float32),
            pltpu.VMEM((2, 4, _CI), jnp.int32),
            pltpu.VMEM((2, 1, 128), jnp.float32),
            pltpu.SemaphoreType.DMA,
            pltpu.SemaphoreType.DMA,
            pltpu.SemaphoreType.DMA,
        ],
        compiler_params=pltpu.CompilerParams(
            use_tc_tiling_on_sc=True, needs_layout_passes=False),
    )
    def body(cur_hbm, cor_hbm, ent_hbm, rel_hbm, out_hbm, sp_hbm,
             row_v, vbuf, idxb, accv, sem, sps0, sps1):
        wid = lax.axis_index("s") * _NC + lax.axis_index("c")
        z16 = jnp.zeros((16,), jnp.int32)

        def idx_pairs(cols, c):
            ds = pl.ds(c * _CI, _CI)
            buf = lax.rem(c, 2)
            return [(src.at[pl.ds(col, 1), ds], idxb.at[buf, pl.ds(j, 1)])
                    for j, (src, col) in enumerate(cols)]

        def start_idx(cols, c):
            for s, t in idx_pairs(cols, c):
                pltpu.async_copy(s, t, sem)

        def wait_idx(cols, c):
            for s, t in idx_pairs(cols, c):
                pltpu.make_async_copy(s, t, sem).wait()

        def spill_pair(c):
            buf = lax.rem(c, 2)
            return (vbuf.at[pl.ds(buf, 1)],
                    sp_hbm.at[pl.ds(wid, 1), pl.ds(0, 2), pl.ds(c * _CI, _CI)])

        def on_spill_sem(c, fn):
            v, h = spill_pair(c)

            @pl.when(lax.rem(c, 2) == 0)
            def _():
                fn(v, h, sps0)

            @pl.when(lax.rem(c, 2) == 1)
            def _():
                fn(v, h, sps1)

        ecols = [(cur_hbm, 0), (cur_hbm, 2), (cor_hbm, 0), (cor_hbm, 2)]
        rcols = [(cur_hbm, 1), (cor_hbm, 1)]

        def pass_body(p, _):
            d = wid + _NW * p

            start_idx(ecols, 0)
            pltpu.sync_copy(ent_hbm.at[pl.ds(d, 1)], row_v)

            def e_chunk(c, _):
                wait_idx(ecols, c)

                @pl.when(c + 1 < _NCH)
                def _():
                    start_idx(ecols, c + 1)

                @pl.when(c >= 2)
                def _():
                    on_spill_sem(c - 2,
                                 lambda v, h, s:
                                 pltpu.make_async_copy(v, h, s).wait())

                buf = lax.rem(c, 2)

                def eb(i, _):
                    for u in range(_UNROLL):
                        off = i * (16 * _UNROLL) + u * 16
                        for l in range(2):
                            hi = idxb[buf, 2 * l, pl.ds(off, 16)]
                            ti = idxb[buf, 2 * l + 1, pl.ds(off, 16)]
                            gh = plsc.load_gather(row_v, [z16, hi])
                            gt = plsc.load_gather(row_v, [z16, ti])
                            vbuf[buf, l, pl.ds(off, 16)] = gh - gt
                    return 0

                lax.fori_loop(0, _CI // (16 * _UNROLL), eb, 0)
                on_spill_sem(c, lambda v, h, s: pltpu.async_copy(v, h, s))
                return 0

            lax.fori_loop(0, _NCH, e_chunk, 0)
            for cc in (_NCH - 2, _NCH - 1):
                on_spill_sem(cc,
                             lambda v, h, s:
                             pltpu.make_async_copy(v, h, s).wait())

            start_idx(rcols, 0)
            for cc in (0, 1):
                on_spill_sem(cc, lambda v, h, s: pltpu.async_copy(h, v, s))
            pltpu.sync_copy(rel_hbm.at[pl.ds(d, 1)], row_v)

            def r_chunk(c, accs):
                wait_idx(rcols, c)

                @pl.when(c + 1 < _NCH)
                def _():
                    start_idx(rcols, c + 1)

                on_spill_sem(c,
                             lambda v, h, s:
                             pltpu.make_async_copy(h, v, s).wait())
                buf = lax.rem(c, 2)

                def rb(i, carry):
                    a0, a1 = carry
                    for u in range(_UNROLL):
                        off = i * (16 * _UNROLL) + u * 16
                        ri = idxb[buf, 0, pl.ds(off, 16)]
                        rci = idxb[buf, 1, pl.ds(off, 16)]
                        v0 = vbuf[buf, 0, pl.ds(off, 16)] \
                            + plsc.load_gather(row_v, [z16, ri])
                        v1 = vbuf[buf, 1, pl.ds(off, 16)] \
                            + plsc.load_gather(row_v, [z16, rci])
                        a0 = a0 + v0 * v0
                        a1 = a1 + v1 * v1
                    return (a0, a1)

                accs = lax.fori_loop(0, _CI // (16 * _UNROLL), rb, accs)

                @pl.when(c + 2 < _NCH)
                def _():
                    on_spill_sem(c + 2,
                                 lambda v, h, s: pltpu.async_copy(h, v, s))

                return accs

            accs = lax.fori_loop(
                0, _NCH, r_chunk,
                (jnp.zeros((16,), jnp.float32), jnp.zeros((16,), jnp.float32)))
            for l in range(2):
                accv[l, 0, pl.ds(0, 16)] = accs[l]
                pltpu.sync_copy(accv.at[pl.ds(l, 1)],
                                out_hbm.at[pl.ds(l, 1), pl.ds(d, 1)])
            return 0

        lax.fori_loop(0, _NPASS, pass_body, 0)

    return body(cur_t, cor_t, ent_t, rel_t)[0]


def _finish(partials):
    def body(p_ref, o_ref):
        p = p_ref[:, :, 0:16]
        s = jnp.sum(p, axis=2)
        dist = jnp.sqrt(s)
        m = jnp.maximum(dist[0:1] - dist[1:2] + _MARGIN, 0.0)
        o_ref[...] = jnp.sum(m, axis=1, keepdims=True) * (1.0 / _DIM)

    return pl.pallas_call(
        body,
        out_shape=jax.ShapeDtypeStruct((1, 1), jnp.float32),
    )(partials)


@jax.jit
def kernel(current_list, corrupt_list, ent_emb, rel_emb):
    partials = _sc_partials(
        current_list.T, corrupt_list.T, ent_emb.T, rel_emb.T)
    loss = _finish(partials)
    return loss[0, 0]


# speedup vs baseline: 1.2556x; 1.2373x over previous
import functools

import jax
import jax.numpy as jnp
from jax import lax
from jax.experimental import pallas as pl
from jax.experimental.pallas import tpu as pltpu
from jax.experimental.pallas import tpu_sc as plsc

_ENTITY_NUM = 100000
_DIM = 64
_MARGIN = 1.0
_BATCH = 16384

_NC = 2
_NS = 16
_NW = _NC * _NS
_NPASS = _DIM // _NW
_CI = 2048
_NCH = _BATCH // _CI
_UNROLL = 8


def _sc_partials(cur_t, cor_t, ent_t, rel_t):
    mesh = plsc.VectorSubcoreMesh(core_axis_name="c", subcore_axis_name="s")

    @functools.partial(
        pl.kernel,
        mesh=mesh,
        out_type=(
            jax.ShapeDtypeStruct((2, _DIM, 128), jnp.float32),
            jax.ShapeDtypeStruct((_NW, 2, _BATCH), jnp.float32),
        ),
        scratch_types=[
            pltpu.VMEM((1, _ENTITY_NUM), jnp.float32),
            pltpu.VMEM((2, 2, _CI), jnp.float32),
            pltpu.VMEM((2, 4, _CI), jnp.int32),
            pltpu.VMEM((2, 1, 128), jnp.float32),
            pltpu.SemaphoreType.DMA,
            pltpu.SemaphoreType.DMA,
            pltpu.SemaphoreType.DMA,
        ],
        compiler_params=pltpu.CompilerParams(
            use_tc_tiling_on_sc=True, needs_layout_passes=False),
    )
    def body(cur_hbm, cor_hbm, ent_hbm, rel_hbm, out_hbm, sp_hbm,
             row_v, vbuf, idxb, accv, sem, sps0, sps1):
        wid = lax.axis_index("s") * _NC + lax.axis_index("c")
        z16 = jnp.zeros((16,), jnp.int32)

        def idx_pairs(cols, c):
            ds = pl.ds(c * _CI, _CI)
            buf = lax.rem(c, 2)
            return [(src.at[pl.ds(col, 1), ds], idxb.at[buf, pl.ds(j, 1)])
                    for j, (src, col) in enumerate(cols)]

        def start_idx(cols, c):
            for s, t in idx_pairs(cols, c):
                pltpu.async_copy(s, t, sem)

        def wait_idx(cols, c):
            for s, t in idx_pairs(cols, c):
                pltpu.make_async_copy(s, t, sem).wait()

        def spill_pair(c):
            buf = lax.rem(c, 2)
            return (vbuf.at[pl.ds(buf, 1)],
                    sp_hbm.at[pl.ds(wid, 1), pl.ds(0, 2), pl.ds(c * _CI, _CI)])

        def on_spill_sem(c, fn):
            v, h = spill_pair(c)

            @pl.when(lax.rem(c, 2) == 0)
            def _():
                fn(v, h, sps0)

            @pl.when(lax.rem(c, 2) == 1)
            def _():
                fn(v, h, sps1)

        ecols = [(cur_hbm, 0), (cur_hbm, 2), (cor_hbm, 0), (cor_hbm, 2)]
        rcols = [(cur_hbm, 1), (cor_hbm, 1)]

        def pass_body(p, _):
            d = wid + _NW * p

            start_idx(ecols, 0)
            pltpu.sync_copy(ent_hbm.at[pl.ds(d, 1)], row_v)

            def e_chunk(c, _):
                wait_idx(ecols, c)

                @pl.when(c + 1 < _NCH)
                def _():
                    start_idx(ecols, c + 1)

                @pl.when(c >= 2)
                def _():
                    on_spill_sem(c - 2,
                                 lambda v, h, s:
                                 pltpu.make_async_copy(v, h, s).wait())

                buf = lax.rem(c, 2)

                @plsc.parallel_loop(0, _CI // 16, unroll=_UNROLL)
                def _(i):
                    off = i * 16
                    for l in range(2):
                        hi = idxb[buf, 2 * l, pl.ds(off, 16)]
                        ti = idxb[buf, 2 * l + 1, pl.ds(off, 16)]
                        gh = plsc.load_gather(row_v, [z16, hi])
                        gt = plsc.load_gather(row_v, [z16, ti])
                        vbuf[buf, l, pl.ds(off, 16)] = gh - gt

                on_spill_sem(c, lambda v, h, s: pltpu.async_copy(v, h, s))
                return 0

            lax.fori_loop(0, _NCH, e_chunk, 0)
            for cc in (_NCH - 2, _NCH - 1):
                on_spill_sem(cc,
                             lambda v, h, s:
                             pltpu.make_async_copy(v, h, s).wait())

            start_idx(rcols, 0)
            for cc in (0, 1):
                on_spill_sem(cc, lambda v, h, s: pltpu.async_copy(h, v, s))
            pltpu.sync_copy(rel_hbm.at[pl.ds(d, 1)], row_v)

            def r_chunk(c, accs):
                wait_idx(rcols, c)

                @pl.when(c + 1 < _NCH)
                def _():
                    start_idx(rcols, c + 1)

                on_spill_sem(c,
                             lambda v, h, s:
                             pltpu.make_async_copy(h, v, s).wait())
                buf = lax.rem(c, 2)

                def rb(i, carry):
                    a0, a1 = carry
                    off = i * 16
                    ri = idxb[buf, 0, pl.ds(off, 16)]
                    rci = idxb[buf, 1, pl.ds(off, 16)]
                    v0 = vbuf[buf, 0, pl.ds(off, 16)] \
                        + plsc.load_gather(row_v, [z16, ri])
                    v1 = vbuf[buf, 1, pl.ds(off, 16)] \
                        + plsc.load_gather(row_v, [z16, rci])
                    return (a0 + v0 * v0, a1 + v1 * v1)

                accs = plsc.parallel_loop(
                    0, _CI // 16, unroll=_UNROLL, carry=accs)(rb)

                @pl.when(c + 2 < _NCH)
                def _():
                    on_spill_sem(c + 2,
                                 lambda v, h, s: pltpu.async_copy(h, v, s))

                return accs

            accs = lax.fori_loop(
                0, _NCH, r_chunk,
                (jnp.zeros((16,), jnp.float32), jnp.zeros((16,), jnp.float32)))
            for l in range(2):
                accv[l, 0, pl.ds(0, 16)] = accs[l]
                pltpu.sync_copy(accv.at[pl.ds(l, 1)],
                                out_hbm.at[pl.ds(l, 1), pl.ds(d, 1)])
            return 0

        lax.fori_loop(0, _NPASS, pass_body, 0)

    return body(cur_t, cor_t, ent_t, rel_t)[0]


def _finish(partials):
    def body(p_ref, o_ref):
        p = p_ref[:, :, 0:16]
        s = jnp.sum(p, axis=2)
        dist = jnp.sqrt(s)
        m = jnp.maximum(dist[0:1] - dist[1:2] + _MARGIN, 0.0)
        o_ref[...] = jnp.sum(m, axis=1, keepdims=True) * (1.0 / _DIM)

    return pl.pallas_call(
        body,
        out_shape=jax.ShapeDtypeStruct((1, 1), jnp.float32),
    )(partials)


@jax.jit
def kernel(current_list, corrupt_list, ent_emb, rel_emb):
    partials = _sc_partials(
        current_list.T, corrupt_list.T, ent_emb.T, rel_emb.T)
    loss = _finish(partials)
    return loss[0, 0]
